# baseline (speedup 1.0000x reference)
import functools

import jax
import jax.numpy as jnp
from jax import lax
from jax.experimental import pallas as pl
from jax.experimental.pallas import tpu as pltpu
from jax.experimental.pallas import tpu_sc as plsc

_DU = 20
_DZ = 5
_N = _DU * _DZ
_GL = 16

_NC, _NS = 2, 16
_NW = _NC * _NS
_RC = 8


def _g1_runs():
    runs = []
    for j in range(_DU):
        off = (_DZ * (j - 1)) % _N
        w = min(3 * _DZ, _N - off)
        runs.append((_N * j + off, w, 0))
        if w < 3 * _DZ:
            runs.append((_N * j, 3 * _DZ - w, w))
    return runs


_G1_RUNS = _g1_runs()
_G1_GRANS = sorted({p // _GL
                    for a, n, _ in _G1_RUNS for p in range(a, a + n)})
_G1_T = {g: t for t, g in enumerate(_G1_GRANS)}
_K1 = len(_G1_GRANS)
_PK1 = _GL * _K1
_G1_ZEROS = [u for u in range(_DU * _N // _GL) if u not in _G1_T]
_g1_pos = lambda p: _GL * _G1_T[p // _GL] + p % _GL


def _tc_body(x_ref, w0, b0, w1, b1, w2, b2, w3, b3,
             f1_ref, f2_ref, p1_ref, g2_ref):
    bb = x_ref.shape[0]
    x = x_ref[...]

    pieces = []
    for j in range(_DU):
        jm, jp = (j - 1) % _DU, (j + 1) % _DU
        h = (x[:, jm:jm + 1] * w0[0:1, :]
             + x[:, j:j + 1] * w0[1:2, :]
             + x[:, jp:jp + 1] * w0[2:3, :]) + b0[...]
        pieces.append(h)
    h = jnp.maximum(jnp.concatenate(pieces, axis=0), 0.0)

    h = jnp.maximum(
        jnp.dot(h, w1[...], preferred_element_type=jnp.float32) + b1[...], 0.0)
    h = jnp.maximum(
        jnp.dot(h, w2[...], preferred_element_type=jnp.float32) + b2[...], 0.0)
    out = jnp.dot(h, w3[...], preferred_element_type=jnp.float32) + b3[...]

    p1_ref[...] = jnp.zeros_like(p1_ref)
    g2_ref[...] = jnp.zeros_like(g2_ref)

    run_by_j = {}
    for a, n, voff in _G1_RUNS:
        run_by_j.setdefault(a // _N, []).append((a, n, voff))

    for j in range(_DU):
        r0, r1 = j * bb, (j + 1) * bb
        s = out[r0:r1, :]

        f1_ref[:, j:j + 1] = s[:, 0:1]
        f2_ref[:, _DZ * j:_DZ * (j + 1)] = s[:, 16:16 + _DZ]

        for a, n, voff in run_by_j[j]:
            ps = _g1_pos(a)
            p1_ref[:, ps:ps + n] = s[:, 1 + voff:1 + voff + n]

        off = (_DZ * (j - 2)) % _N
        w25 = min(5 * _DZ, _N - off)
        for z in range(_DZ):
            c0 = 16 + _DZ + 25 * z
            base = _N * (_DZ * j + z)
            g2_ref[:, base + off:base + off + w25] = s[:, c0:c0 + w25]
            if w25 < 5 * _DZ:
                g2_ref[:, base:base + 5 * _DZ - w25] = s[:, c0 + w25:c0 + 25]


def _sc_g1_body(pack_hbm, g1_hbm, pack_v, row_v):
    n_chunks = pack_hbm.shape[0] // (_NW * _RC)
    wid = lax.axis_index("s") * _NC + lax.axis_index("c")
    zero = jnp.zeros((_GL,), jnp.float32)
    for r in range(_RC):
        for u in _G1_ZEROS:
            row_v[r, _GL * u:_GL * (u + 1)] = zero

    @pl.loop(0, n_chunks)
    def _chunk(i):
        base = wid * (n_chunks * _RC) + i * _RC
        pltpu.sync_copy(pack_hbm.at[pl.ds(base, _RC)], pack_v)
        for r in range(_RC):
            for t, g in enumerate(_G1_GRANS):
                row_v[r, _GL * g:_GL * (g + 1)] = pack_v[r, _GL * t:_GL * (t + 1)]
        pltpu.sync_copy(row_v, g1_hbm.at[pl.ds(base, _RC)])


def kernel(x, w1_0, b1_0, w1_1, b1_1, w1_2, b1_2, w1_3, b1_3,
           w2_0, b2_0, w2_1, b2_1, w2_2, b2_2, w2_3, b2_3):
    batch = x.shape[0]
    bb = 256 if batch % 256 == 0 else batch
    grid = (batch // bb,)
    f32 = jnp.float32

    w0 = jnp.concatenate([w1_0.T, w2_0.T], axis=1)
    b0 = jnp.concatenate([b1_0, b2_0]).reshape(1, -1)
    w1 = jnp.zeros((32, 64), f32).at[:16, :32].set(w1_1.T).at[16:, 32:].set(w2_1.T)
    b1 = jnp.concatenate([b1_1, b2_1]).reshape(1, -1)
    w2 = jnp.zeros((64, 32), f32).at[:32, :16].set(w1_2.T).at[32:, 16:].set(w2_2.T)
    b2 = jnp.concatenate([b1_2, b2_2]).reshape(1, -1)
    w3 = jnp.zeros((32, 146), f32).at[:16, :16].set(w1_3.T).at[16:, 16:].set(w2_3.T)
    b3 = jnp.concatenate([b1_3, b2_3]).reshape(1, -1)
    ws = [w0, b0, w1, b1, w2, b2, w3, b3]

    def wspec(a):
        return pl.BlockSpec(a.shape, lambda i: (0,) * a.ndim)

    f1, f2, g1pack, g2 = pl.pallas_call(
        _tc_body,
        grid=grid,
        in_specs=[pl.BlockSpec((bb, _DU), lambda i: (i, 0))]
                  + [wspec(a) for a in ws],
        out_specs=[
            pl.BlockSpec((bb, _DU), lambda i: (i, 0)),
            pl.BlockSpec((bb, _N), lambda i: (i, 0)),
            pl.BlockSpec((bb, _PK1), lambda i: (i, 0)),
            pl.BlockSpec((bb, _N * _N), lambda i: (i, 0)),
        ],
        out_shape=[
            jax.ShapeDtypeStruct((batch, _DU), x.dtype),
            jax.ShapeDtypeStruct((batch, _N), x.dtype),
            jax.ShapeDtypeStruct((batch, _PK1), x.dtype),
            jax.ShapeDtypeStruct((batch, _N * _N), x.dtype),
        ],
    )(x, *ws)

    g1 = pl.kernel(
        _sc_g1_body,
        out_type=jax.ShapeDtypeStruct((batch, _DU * _N), f32),
        mesh=plsc.VectorSubcoreMesh(core_axis_name="c", subcore_axis_name="s"),
        scratch_types=[
            pltpu.VMEM((_RC, _PK1), f32),
            pltpu.VMEM((_RC, _DU * _N), f32),
        ],
    )(g1pack)

    return (f1.reshape(batch, _DU, 1), g1.reshape(batch, _DU, _N),
            f2.reshape(batch, _N, 1), g2.reshape(batch, _N, _N))
